# baseline (speedup 1.0000x reference)
import functools

import jax
import jax.numpy as jnp
from jax import lax
from jax.experimental import pallas as pl
from jax.experimental.pallas import tpu as pltpu
from jax.experimental.pallas import tpu_sc as plsc

B = 1024
D = 256
CAP = 1000
PK_W = CAP // 2 + 1
K = 10

NC, NS, L = 2, 16, 16
NW = NC * NS
ROWS_PER_W = B // NW
GROUPS = ROWS_PER_W // L
OCT = 8
NBLK = CAP // OCT
IDX_MASK = 127

_DN = (((1,), (1,)), ((), ()))


def _tc_distances(state_ref, w1_ref, b1_ref, w2_ref, b2_ref, mem_ref,
                  out_ref, m8_ref):
    s = state_ref[...]
    w1 = w1_ref[...]
    w2 = w2_ref[...]
    b1 = b1_ref[...]
    b2 = b2_ref[...]
    m = mem_ref[...]
    hp = jax.lax.Precision.DEFAULT
    h = jnp.maximum(
        jax.lax.dot_general(s, w1, _DN, precision=hp,
                            preferred_element_type=jnp.float32) + b1, 0.0)
    e = jax.lax.dot_general(h, w2, _DN, precision=hp,
                            preferred_element_type=jnp.float32) + b2
    emt = jax.lax.dot_general(e, m, _DN, precision=hp,
                              preferred_element_type=jnp.float32)
    mem_sq = jnp.sum(m * m, axis=1)[None, :]
    e_sq = jnp.sum(e * e, axis=1, keepdims=True)
    d2 = jnp.maximum(e_sq - 2.0 * emt + mem_sq, 0.0)
    out_ref[:, :CAP] = d2.astype(jnp.bfloat16)
    mn = jnp.minimum(d2[:, :NBLK], d2[:, NBLK:2 * NBLK])
    for u in range(2, OCT):
        mn = jnp.minimum(mn, d2[:, u * NBLK:(u + 1) * NBLK])
    tag = lax.broadcasted_iota(jnp.int32, (B, NBLK), 1)
    m8_ref[...] = lax.bitcast_convert_type(
        (lax.bitcast_convert_type(mn, jnp.int32) & jnp.int32(~IDX_MASK))
        | tag, jnp.float32)


_tc_call = pl.pallas_call(
    _tc_distances,
    out_shape=(jax.ShapeDtypeStruct((B, 2 * PK_W), jnp.bfloat16),
               jax.ShapeDtypeStruct((B, NBLK), jnp.float32)),
)


def _psqrt(x):
    i = plsc.bitcast(x, jnp.int32)
    y = plsc.bitcast((i >> 1) + 0x1FBD1DF5, jnp.float32)
    for _ in range(2):
        y = 0.5 * (y + x / y)
    return jnp.where(x > 0.0, y, 0.0)


def _insert(ms, v):
    new = [jnp.minimum(ms[0], v)]
    for i in range(1, K):
        new.append(jnp.minimum(ms[i], jnp.maximum(v, ms[i - 1])))
    return tuple(new)


@functools.partial(
    pl.kernel,
    out_type=jax.ShapeDtypeStruct((B,), jnp.float32),
    mesh=plsc.VectorSubcoreMesh(core_axis_name="c", subcore_axis_name="s"),
    compiler_params=pltpu.CompilerParams(needs_layout_passes=False),
    scratch_types=[
        pltpu.VMEM((ROWS_PER_W, PK_W), jnp.int32),
        pltpu.VMEM((ROWS_PER_W, NBLK), jnp.float32),
        pltpu.VMEM((ROWS_PER_W,), jnp.float32),
        pltpu.SemaphoreType.DMA,
    ],
)
def _sc_topk(d2_hbm, m8_hbm, out_hbm, buf_v, m8_v, out_v, sem):
    wid = lax.axis_index("s") * NC + lax.axis_index("c")
    row_base = wid * ROWS_PER_W
    cp = pltpu.async_copy(d2_hbm.at[pl.ds(row_base, ROWS_PER_W), :],
                          buf_v, sem)
    pltpu.sync_copy(m8_hbm.at[pl.ds(row_base, ROWS_PER_W), :], m8_v)

    rows = [lax.iota(jnp.int32, L) + (g * L) for g in range(GROUPS)]
    init = tuple(
        tuple(jnp.full((L,), 1e30, jnp.float32) for _ in range(K))
        for _ in range(GROUPS))
    tv0 = jnp.full((L,), 0, jnp.int32)

    UNROLL = 5

    def body(t, carry):
        tv, mss = carry
        mss = list(mss)
        for u in range(UNROLL):
            for g in range(GROUPS):
                v = plsc.load_gather(m8_v, [rows[g], tv + u])
                mss[g] = _insert(mss[g], v)
        return tv + UNROLL, tuple(mss)

    _, mss = lax.fori_loop(0, NBLK // UNROLL, body, (tv0, init))

    cp.wait()

    init2 = tuple(
        tuple(jnp.full((L,), 1e30, jnp.float32) for _ in range(K))
        for _ in range(GROUPS))

    def body2(k, mss2):
        mss2 = list(mss2)
        for g in range(GROUPS):
            w = mss[g][0]
            for i in range(1, K):
                w = jnp.where(k == i, mss[g][i], w)
            q = plsc.bitcast(w, jnp.int32) & jnp.int32(IDX_MASK)
            for u in range(OCT):
                c = q + (u * NBLK)
                pw = plsc.load_gather(buf_v, [rows[g], c >> 1])
                bits = jnp.where((c & 1) == 1, pw & jnp.int32(-65536),
                                 pw << 16)
                mss2[g] = _insert(mss2[g], plsc.bitcast(bits, jnp.float32))
        return tuple(mss2)

    mss2 = lax.fori_loop(0, K, body2, init2)

    for g in range(GROUPS):
        ms = mss2[g]
        acc = _psqrt(ms[0])
        for i in range(1, K):
            acc = acc + _psqrt(ms[i])
        out_v[pl.ds(g * L, L)] = acc * (1.0 / K)

    pltpu.sync_copy(out_v, out_hbm.at[pl.ds(row_base, ROWS_PER_W)])


def kernel(state, W1, b1, W2, b2, memory):
    d2b, m8 = _tc_call(state, W1, b1[None, :], W2, b2[None, :], memory)
    pk = lax.bitcast_convert_type(d2b.reshape(B, PK_W, 2), jnp.int32)
    return _sc_topk(pk, m8)


# speedup vs baseline: 1.6450x; 1.6450x over previous
import functools

import jax
import jax.numpy as jnp
from jax import lax
from jax.experimental import pallas as pl
from jax.experimental.pallas import tpu as pltpu
from jax.experimental.pallas import tpu_sc as plsc

B = 1024
D = 256
CAP = 1000
STRIDE = 1001
K = 10

NC, NS, L = 2, 16, 16
NW = NC * NS
ROWS_PER_W = B // NW
GROUPS = ROWS_PER_W // L
OCT = 8
NBLK = CAP // OCT
IDX_MASK = 127

_DN = (((1,), (1,)), ((), ()))


def _tc_distances(state_ref, w1_ref, b1_ref, w2_ref, b2_ref, mem_ref,
                  out_ref, m8_ref):
    s = state_ref[...]
    w1 = w1_ref[...]
    w2 = w2_ref[...]
    b1 = b1_ref[...]
    b2 = b2_ref[...]
    m = mem_ref[...]
    hp = jax.lax.Precision.DEFAULT
    h = jnp.maximum(
        jax.lax.dot_general(s, w1, _DN, precision=hp,
                            preferred_element_type=jnp.float32) + b1, 0.0)
    e = jax.lax.dot_general(h, w2, _DN, precision=hp,
                            preferred_element_type=jnp.float32) + b2
    emt = jax.lax.dot_general(e, m, _DN, precision=hp,
                              preferred_element_type=jnp.float32)
    mem_sq = jnp.sum(m * m, axis=1)[None, :]
    e_sq = jnp.sum(e * e, axis=1, keepdims=True)
    d2 = jnp.maximum(e_sq - 2.0 * emt + mem_sq, 0.0)
    out_ref[:, :CAP] = d2
    mn = jnp.minimum(d2[:, :NBLK], d2[:, NBLK:2 * NBLK])
    for u in range(2, OCT):
        mn = jnp.minimum(mn, d2[:, u * NBLK:(u + 1) * NBLK])
    tag = lax.broadcasted_iota(jnp.int32, (B, NBLK), 1)
    m8_ref[...] = lax.bitcast_convert_type(
        (lax.bitcast_convert_type(mn, jnp.int32) & jnp.int32(~IDX_MASK))
        | tag, jnp.float32)


_tc_call = pl.pallas_call(
    _tc_distances,
    out_shape=(jax.ShapeDtypeStruct((B, STRIDE), jnp.float32),
               jax.ShapeDtypeStruct((B, NBLK), jnp.float32)),
)


def _psqrt(x):
    i = plsc.bitcast(x, jnp.int32)
    y = plsc.bitcast((i >> 1) + 0x1FBD1DF5, jnp.float32)
    for _ in range(2):
        y = 0.5 * (y + x / y)
    return jnp.where(x > 0.0, y, 0.0)


def _insert(ms, v):
    new = [jnp.minimum(ms[0], v)]
    for i in range(1, K):
        new.append(jnp.minimum(ms[i], jnp.maximum(v, ms[i - 1])))
    return tuple(new)


@functools.partial(
    pl.kernel,
    out_type=jax.ShapeDtypeStruct((B,), jnp.float32),
    mesh=plsc.VectorSubcoreMesh(core_axis_name="c", subcore_axis_name="s"),
    compiler_params=pltpu.CompilerParams(needs_layout_passes=False),
    scratch_types=[
        pltpu.VMEM((ROWS_PER_W, STRIDE), jnp.float32),
        pltpu.VMEM((ROWS_PER_W, NBLK), jnp.float32),
        pltpu.VMEM((ROWS_PER_W,), jnp.float32),
        pltpu.SemaphoreType.DMA,
    ],
)
def _sc_topk(d2_hbm, m8_hbm, out_hbm, buf_v, m8_v, out_v, sem):
    wid = lax.axis_index("s") * NC + lax.axis_index("c")
    row_base = wid * ROWS_PER_W
    cp = pltpu.async_copy(d2_hbm.at[pl.ds(row_base, ROWS_PER_W), :],
                          buf_v, sem)
    pltpu.sync_copy(m8_hbm.at[pl.ds(row_base, ROWS_PER_W), :], m8_v)

    rows = [lax.iota(jnp.int32, L) + (g * L) for g in range(GROUPS)]
    init = tuple(
        tuple(jnp.full((L,), 1e30, jnp.float32) for _ in range(K))
        for _ in range(GROUPS))
    tv0 = jnp.full((L,), 0, jnp.int32)

    UNROLL = 5

    def body(t, carry):
        tv, mss = carry
        mss = list(mss)
        for u in range(UNROLL):
            for g in range(GROUPS):
                v = plsc.load_gather(m8_v, [rows[g], tv + u])
                mss[g] = _insert(mss[g], v)
        return tv + UNROLL, tuple(mss)

    _, mss = lax.fori_loop(0, NBLK // UNROLL, body, (tv0, init))

    cp.wait()

    init2 = tuple(
        tuple(jnp.full((L,), 1e30, jnp.float32) for _ in range(K))
        for _ in range(GROUPS))

    def body2(k, mss2):
        mss2 = list(mss2)
        for g in range(GROUPS):
            w = mss[g][0]
            for i in range(1, K):
                w = jnp.where(k == i, mss[g][i], w)
            q = plsc.bitcast(w, jnp.int32) & jnp.int32(IDX_MASK)
            for u in range(OCT):
                vv = plsc.load_gather(buf_v, [rows[g], q + (u * NBLK)])
                mss2[g] = _insert(mss2[g], vv)
        return tuple(mss2)

    mss2 = lax.fori_loop(0, K, body2, init2)

    for g in range(GROUPS):
        ms = mss2[g]
        acc = _psqrt(ms[0])
        for i in range(1, K):
            acc = acc + _psqrt(ms[i])
        out_v[pl.ds(g * L, L)] = acc * (1.0 / K)

    pltpu.sync_copy(out_v, out_hbm.at[pl.ds(row_base, ROWS_PER_W)])


def kernel(state, W1, b1, W2, b2, memory):
    d2, m8 = _tc_call(state, W1, b1[None, :], W2, b2[None, :], memory)
    return _sc_topk(d2, m8)
